# baseline (speedup 1.0000x reference)
import functools

import jax
import jax.numpy as jnp
from jax import lax
from jax.experimental import pallas as pl
from jax.experimental.pallas import tpu as pltpu
from jax.experimental.pallas import tpu_sc as plsc

N_NODES = 50000
FEAT = 256
HF = FEAT // 2
N_GRAPHS = 512
C1 = 80
NCH1 = N_NODES // C1
C2 = 80
NCH2 = N_NODES // C2
NC = 2
NS = 16
NW = NC * NS
L = 16
HV = HF // L

_mesh = plsc.VectorSubcoreMesh(core_axis_name="c", subcore_axis_name="s")


def _zero_acc(acc, rows, vregs):
    zero = jnp.zeros((L,), jnp.float32)

    def zrow(r, _):
        for j in range(vregs):
            acc[r, pl.ds(j * L, L)] = zero
        return 0
    lax.fori_loop(0, rows, zrow, 0)


@functools.partial(
    pl.kernel,
    out_type=[
        jax.ShapeDtypeStruct((NW * N_GRAPHS, HF), jnp.float32),
        jax.ShapeDtypeStruct((NW * N_GRAPHS, HF), jnp.float32),
        jax.ShapeDtypeStruct((NW * N_GRAPHS, L), jnp.float32),
    ],
    mesh=_mesh,
    scratch_types=[
        pltpu.VMEM((C1,), jnp.int32),
        pltpu.VMEM((C1, HF), jnp.float32),
        pltpu.VMEM((N_GRAPHS, HF), jnp.float32),
        pltpu.VMEM((32, L), jnp.float32),
        pltpu.SMEM((N_GRAPHS,), jnp.float32),
    ],
)
def _seg_sums_sc(x_hbm, b_hbm, slo_out, shi_out, cnt_out,
                 idx_v, x_v, acc, stage_v, cnt_s):
    cid = lax.axis_index("c")
    sid = lax.axis_index("s")
    wid = sid * NC + cid
    n_i = (NCH1 - wid + NW - 1) // NW
    one = jnp.ones((L,), jnp.float32)
    out_base = wid * N_GRAPHS

    for half in range(2):
        _zero_acc(acc, N_GRAPHS, HV)
        if half == 0:
            def zc(r, _):
                cnt_s[r] = 0.0
                return 0
            lax.fori_loop(0, N_GRAPHS, zc, 0)

        zv = jnp.zeros((L,), jnp.float32)

        def chunk_body(i, _):
            base = (wid + i * NW) * C1
            pltpu.sync_copy(b_hbm.at[pl.ds(base, C1)], idx_v)
            pltpu.sync_copy(
                x_hbm.at[pl.ds(base, C1), pl.ds(half * HF, HF)], x_v)

            def gbody(g, carry):
                cur = carry[0]
                regs = carry[1]
                bvec = idx_v[pl.ds(g * L, L)]
                b0 = bvec[0]
                b15 = bvec[L - 1]
                fast = jnp.logical_and(b0 == cur, b0 == b15)

                gsum = []
                for j in range(HV):
                    vals = [x_v[g * L + l, pl.ds(j * L, L)] for l in range(L)]
                    while len(vals) > 1:
                        vals = [vals[k] + vals[k + 1]
                                for k in range(0, len(vals), 2)]
                    gsum.append(vals[0])

                @pl.when(jnp.logical_not(fast))
                def _():
                    for j in range(HV):
                        acc[cur, pl.ds(j * L, L)] = (
                            acc[cur, pl.ds(j * L, L)] + regs[j])
                    for l in range(L):
                        n = g * L + l
                        b = bvec[l]
                        for j in range(HV):
                            acc[b, pl.ds(j * L, L)] = (
                                acc[b, pl.ds(j * L, L)]
                                + x_v[n, pl.ds(j * L, L)])
                        if half == 0:
                            cnt_s[b] = cnt_s[b] + 1.0

                if half == 0:
                    @pl.when(fast)
                    def _():
                        cnt_s[b0] = cnt_s[b0] + 16.0

                new_regs = tuple(
                    jnp.where(fast, regs[j] + gsum[j], zv) for j in range(HV))
                new_cur = jnp.where(fast, cur, b15)
                return (new_cur, new_regs)

            cur, regs = lax.fori_loop(
                0, C1 // L, gbody,
                (jnp.int32(N_GRAPHS - 1), tuple(zv for _ in range(HV))))
            for j in range(HV):
                acc[cur, pl.ds(j * L, L)] = acc[cur, pl.ds(j * L, L)] + regs[j]
            return 0
        lax.fori_loop(0, n_i, chunk_body, 0)

        dst = slo_out if half == 0 else shi_out
        pltpu.sync_copy(acc, dst.at[pl.ds(out_base, N_GRAPHS)])
        if half == 0:
            for s in range(N_GRAPHS // 32):
                def srow(r, _):
                    stage_v[r, pl.ds(0, L)] = jnp.full((L,), cnt_s[s * 32 + r])
                    return 0
                lax.fori_loop(0, 32, srow, 0)
                pltpu.sync_copy(
                    stage_v, cnt_out.at[pl.ds(out_base + s * 32, 32)])


def _dense_body(slo_ref, shi_ref, cnt_ref, w_ref, tg_ref):
    slo = jnp.sum(slo_ref[...].reshape(NW, N_GRAPHS, HF), axis=0)
    shi = jnp.sum(shi_ref[...].reshape(NW, N_GRAPHS, HF), axis=0)
    c = jnp.sum(cnt_ref[...].reshape(NW, N_GRAPHS, L),
                axis=(0, 2), keepdims=False).reshape(N_GRAPHS, 1) / L
    s = jnp.concatenate([slo, shi], axis=1)
    mean = s / jnp.clip(c, 1.0, None)
    tg_ref[...] = jnp.tanh(
        jnp.dot(mean, w_ref[...], preferred_element_type=jnp.float32))


def _dense_stage(slo, shi, cnt, w):
    return pl.pallas_call(
        _dense_body,
        out_shape=jax.ShapeDtypeStruct((N_GRAPHS, FEAT), jnp.float32),
    )(slo, shi, cnt, w)


TCB = 1000


def _coef_body(x_ref, b_ref, tg_ref, o_ref):
    xb = x_ref[...]
    m = lax.dot_general(xb, tg_ref[...],
                        dimension_numbers=(((1,), (1,)), ((), ())),
                        preferred_element_type=jnp.float32)
    gids = lax.broadcasted_iota(jnp.int32, (TCB, N_GRAPHS), 1)
    oh = (gids == b_ref[0, 0, :].reshape(TCB, 1)).astype(jnp.float32)
    d = jnp.sum(m * oh, axis=1, keepdims=True)
    coef = 1.0 / (1.0 + jnp.exp(d * -10.0))
    o_ref[...] = jnp.broadcast_to(coef, (TCB, L))


def _coef_stage(x, b3, tg):
    grid = N_NODES // TCB
    return pl.pallas_call(
        _coef_body,
        grid=(grid,),
        in_specs=[
            pl.BlockSpec((TCB, FEAT), lambda i: (i, 0)),
            pl.BlockSpec((1, 1, TCB), lambda i: (i, 0, 0)),
            pl.BlockSpec((N_GRAPHS, FEAT), lambda i: (0, 0)),
        ],
        out_specs=pl.BlockSpec((TCB, L), lambda i: (i, 0)),
        out_shape=jax.ShapeDtypeStruct((N_NODES, L), jnp.float32),
    )(x, b3, tg)


@functools.partial(
    pl.kernel,
    out_type=[
        jax.ShapeDtypeStruct((NW * N_GRAPHS, HF), jnp.float32),
        jax.ShapeDtypeStruct((NW * N_GRAPHS, HF), jnp.float32),
    ],
    mesh=_mesh,
    scratch_types=[
        pltpu.VMEM((C1,), jnp.int32),
        pltpu.VMEM((C1, HF), jnp.float32),
        pltpu.VMEM((C1, L), jnp.float32),
        pltpu.VMEM((N_GRAPHS, HF), jnp.float32),
    ],
)
def _wsum_sc(w_hbm, c_hbm, b_hbm, olo_out, ohi_out, idx_v, x_v, ca_v, acc):
    cid = lax.axis_index("c")
    sid = lax.axis_index("s")
    wid = sid * NC + cid
    n_i = (NCH1 - wid + NW - 1) // NW
    out_base = wid * N_GRAPHS

    for half in range(2):
        _zero_acc(acc, N_GRAPHS, HV)

        zv = jnp.zeros((L,), jnp.float32)

        def chunk_body(i, _):
            base = (wid + i * NW) * C1
            pltpu.sync_copy(b_hbm.at[pl.ds(base, C1)], idx_v)
            pltpu.sync_copy(
                w_hbm.at[pl.ds(base, C1), pl.ds(half * HF, HF)], x_v)
            pltpu.sync_copy(c_hbm.at[pl.ds(base, C1)], ca_v)

            def gbody(g, carry):
                cur = carry[0]
                regs = carry[1]
                bvec = idx_v[pl.ds(g * L, L)]
                b0 = bvec[0]
                b15 = bvec[L - 1]
                fast = jnp.logical_and(b0 == cur, b0 == b15)

                cvs = [ca_v[g * L + l, pl.ds(0, L)] for l in range(L)]
                gsum = []
                for j in range(HV):
                    vals = [cvs[l] * x_v[g * L + l, pl.ds(j * L, L)]
                            for l in range(L)]
                    while len(vals) > 1:
                        vals = [vals[k] + vals[k + 1]
                                for k in range(0, len(vals), 2)]
                    gsum.append(vals[0])

                @pl.when(jnp.logical_not(fast))
                def _():
                    for j in range(HV):
                        acc[cur, pl.ds(j * L, L)] = (
                            acc[cur, pl.ds(j * L, L)] + regs[j])
                    for l in range(L):
                        n = g * L + l
                        b = bvec[l]
                        for j in range(HV):
                            acc[b, pl.ds(j * L, L)] = (
                                acc[b, pl.ds(j * L, L)]
                                + cvs[l] * x_v[n, pl.ds(j * L, L)])

                new_regs = tuple(
                    jnp.where(fast, regs[j] + gsum[j], zv) for j in range(HV))
                new_cur = jnp.where(fast, cur, b15)
                return (new_cur, new_regs)

            cur, regs = lax.fori_loop(
                0, C1 // L, gbody,
                (jnp.int32(N_GRAPHS - 1), tuple(zv for _ in range(HV))))
            for j in range(HV):
                acc[cur, pl.ds(j * L, L)] = acc[cur, pl.ds(j * L, L)] + regs[j]
            return 0
        lax.fori_loop(0, n_i, chunk_body, 0)

        dst = olo_out if half == 0 else ohi_out
        pltpu.sync_copy(acc, dst.at[pl.ds(out_base, N_GRAPHS)])


def _combine_body(olo_ref, ohi_ref, o_ref):
    olo = jnp.sum(olo_ref[...].reshape(NW, N_GRAPHS, HF), axis=0)
    ohi = jnp.sum(ohi_ref[...].reshape(NW, N_GRAPHS, HF), axis=0)
    o_ref[...] = jnp.concatenate([olo, ohi], axis=1)


def _combine(olo, ohi):
    return pl.pallas_call(
        _combine_body,
        out_shape=jax.ShapeDtypeStruct((N_GRAPHS, FEAT), jnp.float32),
    )(olo, ohi)


def kernel(x, batch, W):
    b32 = batch.astype(jnp.int32)
    slo, shi, cnt = _seg_sums_sc(x, b32)
    tg = _dense_stage(slo, shi, cnt, W)
    b3 = b32.reshape(N_NODES // TCB, 1, TCB)
    coefs = _coef_stage(x, b3, tg)
    olo, ohi = _wsum_sc(x, coefs, b32)
    return _combine(olo, ohi)


# speedup vs baseline: 1.0504x; 1.0504x over previous
import functools

import jax
import jax.numpy as jnp
from jax import lax
from jax.experimental import pallas as pl
from jax.experimental.pallas import tpu as pltpu
from jax.experimental.pallas import tpu_sc as plsc

N_NODES = 50000
FEAT = 256
HF = FEAT // 2
N_GRAPHS = 512
C1 = 80
NCH1 = N_NODES // C1
C2 = 80
NCH2 = N_NODES // C2
NC = 2
NS = 16
NW = NC * NS
L = 16
HV = HF // L

_mesh = plsc.VectorSubcoreMesh(core_axis_name="c", subcore_axis_name="s")


def _zero_acc(acc, rows, vregs):
    zero = jnp.zeros((L,), jnp.float32)

    def zrow(r, _):
        for j in range(vregs):
            acc[r, pl.ds(j * L, L)] = zero
        return 0
    lax.fori_loop(0, rows, zrow, 0)


@functools.partial(
    pl.kernel,
    out_type=[
        jax.ShapeDtypeStruct((NW * N_GRAPHS, HF), jnp.float32),
        jax.ShapeDtypeStruct((NW * N_GRAPHS, HF), jnp.float32),
        jax.ShapeDtypeStruct((NW * N_GRAPHS, L), jnp.float32),
    ],
    mesh=_mesh,
    scratch_types=[
        pltpu.VMEM((C1,), jnp.int32),
        pltpu.VMEM((C1, HF), jnp.float32),
        pltpu.VMEM((N_GRAPHS, HF), jnp.float32),
        pltpu.VMEM((32, L), jnp.float32),
        pltpu.SMEM((N_GRAPHS,), jnp.float32),
    ],
)
def _seg_sums_sc(x_hbm, b_hbm, slo_out, shi_out, cnt_out,
                 idx_v, x_v, acc, stage_v, cnt_s):
    cid = lax.axis_index("c")
    sid = lax.axis_index("s")
    wid = sid * NC + cid
    n_i = (NCH1 - wid + NW - 1) // NW
    one = jnp.ones((L,), jnp.float32)
    out_base = wid * N_GRAPHS

    for half in range(2):
        _zero_acc(acc, N_GRAPHS, HV)
        if half == 0:
            def zc(r, _):
                cnt_s[r] = 0.0
                return 0
            lax.fori_loop(0, N_GRAPHS, zc, 0)

        zv = jnp.zeros((L,), jnp.float32)

        def chunk_body(i, _):
            base = (wid + i * NW) * C1
            pltpu.sync_copy(b_hbm.at[pl.ds(base, C1)], idx_v)
            pltpu.sync_copy(
                x_hbm.at[pl.ds(base, C1), pl.ds(half * HF, HF)], x_v)

            def gbody(g, carry):
                cur = carry[0]
                regs = carry[1]
                bvec = idx_v[pl.ds(g * L, L)]
                b0 = bvec[0]
                b15 = bvec[L - 1]
                fast = jnp.logical_and(b0 == cur, b0 == b15)

                gsum = []
                for j in range(HV):
                    vals = [x_v[g * L + l, pl.ds(j * L, L)] for l in range(L)]
                    while len(vals) > 1:
                        vals = [vals[k] + vals[k + 1]
                                for k in range(0, len(vals), 2)]
                    gsum.append(vals[0])

                @pl.when(jnp.logical_not(fast))
                def _():
                    for j in range(HV):
                        acc[cur, pl.ds(j * L, L)] = (
                            acc[cur, pl.ds(j * L, L)] + regs[j])
                    for l in range(L):
                        n = g * L + l
                        b = bvec[l]
                        for j in range(HV):
                            acc[b, pl.ds(j * L, L)] = (
                                acc[b, pl.ds(j * L, L)]
                                + x_v[n, pl.ds(j * L, L)])
                        if half == 0:
                            cnt_s[b] = cnt_s[b] + 1.0

                if half == 0:
                    @pl.when(fast)
                    def _():
                        cnt_s[b0] = cnt_s[b0] + 16.0

                new_regs = tuple(
                    jnp.where(fast, regs[j] + gsum[j], zv) for j in range(HV))
                new_cur = jnp.where(fast, cur, b15)
                return (new_cur, new_regs)

            cur, regs = lax.fori_loop(
                0, C1 // L, gbody,
                (jnp.int32(N_GRAPHS - 1), tuple(zv for _ in range(HV))))
            for j in range(HV):
                acc[cur, pl.ds(j * L, L)] = acc[cur, pl.ds(j * L, L)] + regs[j]
            return 0
        lax.fori_loop(0, n_i, chunk_body, 0)

        dst = slo_out if half == 0 else shi_out
        pltpu.sync_copy(acc, dst.at[pl.ds(out_base, N_GRAPHS)])
        if half == 0:
            for s in range(N_GRAPHS // 32):
                def srow(r, _):
                    stage_v[r, pl.ds(0, L)] = jnp.full((L,), cnt_s[s * 32 + r])
                    return 0
                lax.fori_loop(0, 32, srow, 0)
                pltpu.sync_copy(
                    stage_v, cnt_out.at[pl.ds(out_base + s * 32, 32)])


def _dense_body(slo_ref, shi_ref, cnt_ref, w_ref, tg_ref):
    slo = jnp.sum(slo_ref[...].reshape(NW, N_GRAPHS, HF), axis=0)
    shi = jnp.sum(shi_ref[...].reshape(NW, N_GRAPHS, HF), axis=0)
    c = jnp.sum(cnt_ref[...].reshape(NW, N_GRAPHS, L),
                axis=(0, 2), keepdims=False).reshape(N_GRAPHS, 1) / L
    s = jnp.concatenate([slo, shi], axis=1)
    mean = s / jnp.clip(c, 1.0, None)
    tg_ref[...] = jnp.tanh(
        jnp.dot(mean, w_ref[...], preferred_element_type=jnp.float32))


def _dense_stage(slo, shi, cnt, w):
    return pl.pallas_call(
        _dense_body,
        out_shape=jax.ShapeDtypeStruct((N_GRAPHS, FEAT), jnp.float32),
    )(slo, shi, cnt, w)


TCB = 1000


def _coef_body(x_ref, b_ref, tg_ref, o_ref):
    xb = x_ref[...]
    m = lax.dot_general(xb, tg_ref[...],
                        dimension_numbers=(((1,), (1,)), ((), ())),
                        precision=lax.Precision.HIGHEST,
                        preferred_element_type=jnp.float32)
    gids = lax.broadcasted_iota(jnp.int32, (TCB, N_GRAPHS), 1)
    oh = (gids == b_ref[0, 0, :].reshape(TCB, 1)).astype(jnp.float32)
    d = jnp.sum(m * oh, axis=1, keepdims=True)
    coef = 1.0 / (1.0 + jnp.exp(d * -10.0))
    o_ref[...] = coef * xb


def _coef_stage(x, b3, tg):
    grid = N_NODES // TCB
    return pl.pallas_call(
        _coef_body,
        grid=(grid,),
        in_specs=[
            pl.BlockSpec((TCB, FEAT), lambda i: (i, 0)),
            pl.BlockSpec((1, 1, TCB), lambda i: (i, 0, 0)),
            pl.BlockSpec((N_GRAPHS, FEAT), lambda i: (0, 0)),
        ],
        out_specs=pl.BlockSpec((TCB, FEAT), lambda i: (i, 0)),
        out_shape=jax.ShapeDtypeStruct((N_NODES, FEAT), jnp.float32),
    )(x, b3, tg)


@functools.partial(
    pl.kernel,
    out_type=[
        jax.ShapeDtypeStruct((NW * N_GRAPHS, HF), jnp.float32),
        jax.ShapeDtypeStruct((NW * N_GRAPHS, HF), jnp.float32),
    ],
    mesh=_mesh,
    scratch_types=[
        pltpu.VMEM((C1,), jnp.int32),
        pltpu.VMEM((C1, HF), jnp.float32),
        pltpu.VMEM((N_GRAPHS, HF), jnp.float32),
    ],
)
def _wsum_sc(w_hbm, b_hbm, olo_out, ohi_out, idx_v, x_v, acc):
    cid = lax.axis_index("c")
    sid = lax.axis_index("s")
    wid = sid * NC + cid
    n_i = (NCH1 - wid + NW - 1) // NW
    out_base = wid * N_GRAPHS

    for half in range(2):
        _zero_acc(acc, N_GRAPHS, HV)

        zv = jnp.zeros((L,), jnp.float32)

        def chunk_body(i, _):
            base = (wid + i * NW) * C1
            pltpu.sync_copy(b_hbm.at[pl.ds(base, C1)], idx_v)
            pltpu.sync_copy(
                w_hbm.at[pl.ds(base, C1), pl.ds(half * HF, HF)], x_v)

            def gbody(g, carry):
                cur = carry[0]
                regs = carry[1]
                bvec = idx_v[pl.ds(g * L, L)]
                b0 = bvec[0]
                b15 = bvec[L - 1]
                fast = jnp.logical_and(b0 == cur, b0 == b15)

                gsum = []
                for j in range(HV):
                    vals = [x_v[g * L + l, pl.ds(j * L, L)] for l in range(L)]
                    while len(vals) > 1:
                        vals = [vals[k] + vals[k + 1]
                                for k in range(0, len(vals), 2)]
                    gsum.append(vals[0])

                @pl.when(jnp.logical_not(fast))
                def _():
                    for j in range(HV):
                        acc[cur, pl.ds(j * L, L)] = (
                            acc[cur, pl.ds(j * L, L)] + regs[j])
                    for l in range(L):
                        n = g * L + l
                        b = bvec[l]
                        for j in range(HV):
                            acc[b, pl.ds(j * L, L)] = (
                                acc[b, pl.ds(j * L, L)]
                                + x_v[n, pl.ds(j * L, L)])

                new_regs = tuple(
                    jnp.where(fast, regs[j] + gsum[j], zv) for j in range(HV))
                new_cur = jnp.where(fast, cur, b15)
                return (new_cur, new_regs)

            cur, regs = lax.fori_loop(
                0, C1 // L, gbody,
                (jnp.int32(N_GRAPHS - 1), tuple(zv for _ in range(HV))))
            for j in range(HV):
                acc[cur, pl.ds(j * L, L)] = acc[cur, pl.ds(j * L, L)] + regs[j]
            return 0
        lax.fori_loop(0, n_i, chunk_body, 0)

        dst = olo_out if half == 0 else ohi_out
        pltpu.sync_copy(acc, dst.at[pl.ds(out_base, N_GRAPHS)])


def _combine_body(olo_ref, ohi_ref, o_ref):
    olo = jnp.sum(olo_ref[...].reshape(NW, N_GRAPHS, HF), axis=0)
    ohi = jnp.sum(ohi_ref[...].reshape(NW, N_GRAPHS, HF), axis=0)
    o_ref[...] = jnp.concatenate([olo, ohi], axis=1)


def _combine(olo, ohi):
    return pl.pallas_call(
        _combine_body,
        out_shape=jax.ShapeDtypeStruct((N_GRAPHS, FEAT), jnp.float32),
    )(olo, ohi)


def kernel(x, batch, W):
    b32 = batch.astype(jnp.int32)
    slo, shi, cnt = _seg_sums_sc(x, b32)
    tg = _dense_stage(slo, shi, cnt, W)
    b3 = b32.reshape(N_NODES // TCB, 1, TCB)
    wrows = _coef_stage(x, b3, tg)
    olo, ohi = _wsum_sc(wrows, b32)
    return _combine(olo, ohi)


# speedup vs baseline: 1.1656x; 1.1096x over previous
import functools

import jax
import jax.numpy as jnp
from jax import lax
from jax.experimental import pallas as pl
from jax.experimental.pallas import tpu as pltpu
from jax.experimental.pallas import tpu_sc as plsc

N_NODES = 50000
FEAT = 256
HF = FEAT // 2
N_GRAPHS = 512
C1 = 80
NCH1 = N_NODES // C1
C2 = 80
NCH2 = N_NODES // C2
NC = 2
NS = 16
NW = NC * NS
L = 16
HV = HF // L

_mesh = plsc.VectorSubcoreMesh(core_axis_name="c", subcore_axis_name="s")


def _zero_acc(acc, rows, vregs):
    zero = jnp.zeros((L,), jnp.float32)

    def zrow(r, _):
        for j in range(vregs):
            acc[r, pl.ds(j * L, L)] = zero
        return 0
    lax.fori_loop(0, rows, zrow, 0)


@functools.partial(
    pl.kernel,
    out_type=[
        jax.ShapeDtypeStruct((NW * N_GRAPHS, HF), jnp.float32),
        jax.ShapeDtypeStruct((NW * N_GRAPHS, HF), jnp.float32),
        jax.ShapeDtypeStruct((NW * N_GRAPHS, L), jnp.float32),
    ],
    mesh=_mesh,
    scratch_types=[
        pltpu.VMEM((C1,), jnp.int32),
        pltpu.VMEM((C1, HF), jnp.float32),
        pltpu.VMEM((N_GRAPHS, HF), jnp.float32),
        pltpu.VMEM((32, L), jnp.float32),
        pltpu.SMEM((N_GRAPHS,), jnp.float32),
    ],
)
def _seg_sums_sc(x_hbm, b_hbm, slo_out, shi_out, cnt_out,
                 idx_v, x_v, acc, stage_v, cnt_s):
    cid = lax.axis_index("c")
    sid = lax.axis_index("s")
    wid = sid * NC + cid
    n_i = (NCH1 - wid + NW - 1) // NW
    one = jnp.ones((L,), jnp.float32)
    out_base = wid * N_GRAPHS

    for half in range(2):
        _zero_acc(acc, N_GRAPHS, HV)
        if half == 0:
            def zc(r, _):
                cnt_s[r] = 0.0
                return 0
            lax.fori_loop(0, N_GRAPHS, zc, 0)

        zv = jnp.zeros((L,), jnp.float32)

        def chunk_body(i, _):
            base = (wid + i * NW) * C1
            pltpu.sync_copy(b_hbm.at[pl.ds(base, C1)], idx_v)
            pltpu.sync_copy(
                x_hbm.at[pl.ds(base, C1), pl.ds(half * HF, HF)], x_v)

            def gbody(g, carry):
                cur = carry[0]
                regs = carry[1]
                bvec = idx_v[pl.ds(g * L, L)]
                b0 = bvec[0]
                b15 = bvec[L - 1]
                fast = jnp.logical_and(b0 == cur, b0 == b15)

                gsum = []
                for j in range(HV):
                    vals = [x_v[g * L + l, pl.ds(j * L, L)] for l in range(L)]
                    while len(vals) > 1:
                        vals = [vals[k] + vals[k + 1]
                                for k in range(0, len(vals), 2)]
                    gsum.append(vals[0])

                @pl.when(jnp.logical_not(fast))
                def _():
                    for j in range(HV):
                        acc[cur, pl.ds(j * L, L)] = (
                            acc[cur, pl.ds(j * L, L)] + regs[j])
                    for l in range(L):
                        n = g * L + l
                        b = bvec[l]
                        for j in range(HV):
                            acc[b, pl.ds(j * L, L)] = (
                                acc[b, pl.ds(j * L, L)]
                                + x_v[n, pl.ds(j * L, L)])
                        if half == 0:
                            cnt_s[b] = cnt_s[b] + 1.0

                if half == 0:
                    @pl.when(fast)
                    def _():
                        cnt_s[b0] = cnt_s[b0] + 16.0

                new_regs = tuple(
                    jnp.where(fast, regs[j] + gsum[j], zv) for j in range(HV))
                new_cur = jnp.where(fast, cur, b15)
                return (new_cur, new_regs)

            cur, regs = lax.fori_loop(
                0, C1 // L, gbody,
                (jnp.int32(N_GRAPHS - 1), tuple(zv for _ in range(HV))))
            for j in range(HV):
                acc[cur, pl.ds(j * L, L)] = acc[cur, pl.ds(j * L, L)] + regs[j]
            return 0
        lax.fori_loop(0, n_i, chunk_body, 0)

        dst = slo_out if half == 0 else shi_out
        pltpu.sync_copy(acc, dst.at[pl.ds(out_base, N_GRAPHS)])
        if half == 0:
            for s in range(N_GRAPHS // 32):
                def srow(r, _):
                    stage_v[r, pl.ds(0, L)] = jnp.full((L,), cnt_s[s * 32 + r])
                    return 0
                lax.fori_loop(0, 32, srow, 0)
                pltpu.sync_copy(
                    stage_v, cnt_out.at[pl.ds(out_base + s * 32, 32)])


def _dense_body(slo_ref, shi_ref, cnt_ref, w_ref, tg_ref):
    slo = jnp.sum(slo_ref[...].reshape(NW, N_GRAPHS, HF), axis=0)
    shi = jnp.sum(shi_ref[...].reshape(NW, N_GRAPHS, HF), axis=0)
    c = jnp.sum(cnt_ref[...].reshape(NW, N_GRAPHS, L),
                axis=(0, 2), keepdims=False).reshape(N_GRAPHS, 1) / L
    s = jnp.concatenate([slo, shi], axis=1)
    mean = s / jnp.clip(c, 1.0, None)
    tg_ref[...] = jnp.tanh(
        jnp.dot(mean, w_ref[...], preferred_element_type=jnp.float32))


def _dense_stage(slo, shi, cnt, w):
    return pl.pallas_call(
        _dense_body,
        out_shape=jax.ShapeDtypeStruct((N_GRAPHS, FEAT), jnp.float32),
    )(slo, shi, cnt, w)


TCB = 1000


def _coef_body(x_ref, b_ref, tg_ref, o_ref):
    xb = x_ref[...]
    gids = lax.broadcasted_iota(jnp.int32, (TCB, N_GRAPHS), 1)
    oh = (gids == b_ref[0, 0, :].reshape(TCB, 1)).astype(jnp.float32)
    tgn = lax.dot_general(oh, tg_ref[...],
                          dimension_numbers=(((1,), (0,)), ((), ())),
                          preferred_element_type=jnp.float32)
    d = jnp.sum(xb * tgn, axis=1, keepdims=True)
    coef = 1.0 / (1.0 + jnp.exp(d * -10.0))
    o_ref[...] = coef * xb


def _coef_stage(x, b3, tg):
    grid = N_NODES // TCB
    return pl.pallas_call(
        _coef_body,
        grid=(grid,),
        in_specs=[
            pl.BlockSpec((TCB, FEAT), lambda i: (i, 0)),
            pl.BlockSpec((1, 1, TCB), lambda i: (i, 0, 0)),
            pl.BlockSpec((N_GRAPHS, FEAT), lambda i: (0, 0)),
        ],
        out_specs=pl.BlockSpec((TCB, FEAT), lambda i: (i, 0)),
        out_shape=jax.ShapeDtypeStruct((N_NODES, FEAT), jnp.float32),
    )(x, b3, tg)


@functools.partial(
    pl.kernel,
    out_type=[
        jax.ShapeDtypeStruct((NW * N_GRAPHS, HF), jnp.float32),
        jax.ShapeDtypeStruct((NW * N_GRAPHS, HF), jnp.float32),
    ],
    mesh=_mesh,
    scratch_types=[
        pltpu.VMEM((C1,), jnp.int32),
        pltpu.VMEM((C1, HF), jnp.float32),
        pltpu.VMEM((N_GRAPHS, HF), jnp.float32),
    ],
)
def _wsum_sc(w_hbm, b_hbm, olo_out, ohi_out, idx_v, x_v, acc):
    cid = lax.axis_index("c")
    sid = lax.axis_index("s")
    wid = sid * NC + cid
    n_i = (NCH1 - wid + NW - 1) // NW
    out_base = wid * N_GRAPHS

    for half in range(2):
        _zero_acc(acc, N_GRAPHS, HV)

        zv = jnp.zeros((L,), jnp.float32)

        def chunk_body(i, _):
            base = (wid + i * NW) * C1
            pltpu.sync_copy(b_hbm.at[pl.ds(base, C1)], idx_v)
            pltpu.sync_copy(
                w_hbm.at[pl.ds(base, C1), pl.ds(half * HF, HF)], x_v)

            def gbody(g, carry):
                cur = carry[0]
                regs = carry[1]
                bvec = idx_v[pl.ds(g * L, L)]
                b0 = bvec[0]
                b15 = bvec[L - 1]
                fast = jnp.logical_and(b0 == cur, b0 == b15)

                gsum = []
                for j in range(HV):
                    vals = [x_v[g * L + l, pl.ds(j * L, L)] for l in range(L)]
                    while len(vals) > 1:
                        vals = [vals[k] + vals[k + 1]
                                for k in range(0, len(vals), 2)]
                    gsum.append(vals[0])

                @pl.when(jnp.logical_not(fast))
                def _():
                    for j in range(HV):
                        acc[cur, pl.ds(j * L, L)] = (
                            acc[cur, pl.ds(j * L, L)] + regs[j])
                    for l in range(L):
                        n = g * L + l
                        b = bvec[l]
                        for j in range(HV):
                            acc[b, pl.ds(j * L, L)] = (
                                acc[b, pl.ds(j * L, L)]
                                + x_v[n, pl.ds(j * L, L)])

                new_regs = tuple(
                    jnp.where(fast, regs[j] + gsum[j], zv) for j in range(HV))
                new_cur = jnp.where(fast, cur, b15)
                return (new_cur, new_regs)

            cur, regs = lax.fori_loop(
                0, C1 // L, gbody,
                (jnp.int32(N_GRAPHS - 1), tuple(zv for _ in range(HV))))
            for j in range(HV):
                acc[cur, pl.ds(j * L, L)] = acc[cur, pl.ds(j * L, L)] + regs[j]
            return 0
        lax.fori_loop(0, n_i, chunk_body, 0)

        dst = olo_out if half == 0 else ohi_out
        pltpu.sync_copy(acc, dst.at[pl.ds(out_base, N_GRAPHS)])


def _combine_body(olo_ref, ohi_ref, o_ref):
    olo = jnp.sum(olo_ref[...].reshape(NW, N_GRAPHS, HF), axis=0)
    ohi = jnp.sum(ohi_ref[...].reshape(NW, N_GRAPHS, HF), axis=0)
    o_ref[...] = jnp.concatenate([olo, ohi], axis=1)


def _combine(olo, ohi):
    return pl.pallas_call(
        _combine_body,
        out_shape=jax.ShapeDtypeStruct((N_GRAPHS, FEAT), jnp.float32),
    )(olo, ohi)


def kernel(x, batch, W):
    b32 = batch.astype(jnp.int32)
    slo, shi, cnt = _seg_sums_sc(x, b32)
    tg = _dense_stage(slo, shi, cnt, W)
    b3 = b32.reshape(N_NODES // TCB, 1, TCB)
    wrows = _coef_stage(x, b3, tg)
    olo, ohi = _wsum_sc(wrows, b32)
    return _combine(olo, ohi)


# speedup vs baseline: 1.1710x; 1.0046x over previous
import functools

import jax
import jax.numpy as jnp
from jax import lax
from jax.experimental import pallas as pl
from jax.experimental.pallas import tpu as pltpu
from jax.experimental.pallas import tpu_sc as plsc

N_NODES = 50000
FEAT = 256
HF = FEAT // 2
N_GRAPHS = 512
C1 = 80
NCH1 = N_NODES // C1
C2 = 80
NCH2 = N_NODES // C2
NC = 2
NS = 16
NW = NC * NS
L = 16
HV = HF // L

_mesh = plsc.VectorSubcoreMesh(core_axis_name="c", subcore_axis_name="s")


def _zero_acc(acc, rows, vregs):
    zero = jnp.zeros((L,), jnp.float32)

    def zrow(r, _):
        for j in range(vregs):
            acc[r, pl.ds(j * L, L)] = zero
        return 0
    lax.fori_loop(0, rows, zrow, 0)


@functools.partial(
    pl.kernel,
    out_type=[
        jax.ShapeDtypeStruct((NW * N_GRAPHS, HF), jnp.float32),
        jax.ShapeDtypeStruct((NW * N_GRAPHS, HF), jnp.float32),
        jax.ShapeDtypeStruct((NW * N_GRAPHS, L), jnp.float32),
    ],
    mesh=_mesh,
    scratch_types=[
        pltpu.VMEM((C1,), jnp.int32),
        pltpu.VMEM((C1, HF), jnp.float32),
        pltpu.VMEM((N_GRAPHS, HF), jnp.float32),
        pltpu.VMEM((32, L), jnp.float32),
        pltpu.SMEM((N_GRAPHS,), jnp.float32),
    ],
)
def _seg_sums_sc(x_hbm, b_hbm, slo_out, shi_out, cnt_out,
                 idx_v, x_v, acc, stage_v, cnt_s):
    cid = lax.axis_index("c")
    sid = lax.axis_index("s")
    wid = sid * NC + cid
    n_i = (NCH1 - wid + NW - 1) // NW
    one = jnp.ones((L,), jnp.float32)
    out_base = wid * N_GRAPHS

    for half in range(2):
        _zero_acc(acc, N_GRAPHS, HV)
        if half == 0:
            def zc(r, _):
                cnt_s[r] = 0.0
                return 0
            lax.fori_loop(0, N_GRAPHS, zc, 0)

        zv = jnp.zeros((L,), jnp.float32)

        def chunk_body(i, _):
            base = (wid + i * NW) * C1
            pltpu.sync_copy(b_hbm.at[pl.ds(base, C1)], idx_v)
            pltpu.sync_copy(
                x_hbm.at[pl.ds(base, C1), pl.ds(half * HF, HF)], x_v)

            def gbody(g, carry):
                cur = carry[0]
                regs = carry[1]
                bvec = idx_v[pl.ds(g * L, L)]
                b0 = bvec[0]
                b15 = bvec[L - 1]
                fast = jnp.logical_and(b0 == cur, b0 == b15)

                gsum = []
                for j in range(HV):
                    vals = [x_v[g * L + l, pl.ds(j * L, L)] for l in range(L)]
                    while len(vals) > 1:
                        vals = [vals[k] + vals[k + 1]
                                for k in range(0, len(vals), 2)]
                    gsum.append(vals[0])

                @pl.when(jnp.logical_not(fast))
                def _():
                    for j in range(HV):
                        acc[cur, pl.ds(j * L, L)] = (
                            acc[cur, pl.ds(j * L, L)] + regs[j])
                    for l in range(L):
                        n = g * L + l
                        b = bvec[l]
                        for j in range(HV):
                            acc[b, pl.ds(j * L, L)] = (
                                acc[b, pl.ds(j * L, L)]
                                + x_v[n, pl.ds(j * L, L)])
                        if half == 0:
                            cnt_s[b] = cnt_s[b] + 1.0

                if half == 0:
                    @pl.when(fast)
                    def _():
                        cnt_s[b0] = cnt_s[b0] + 16.0

                new_regs = tuple(
                    jnp.where(fast, regs[j] + gsum[j], zv) for j in range(HV))
                new_cur = jnp.where(fast, cur, b15)
                return (new_cur, new_regs)

            cur, regs = lax.fori_loop(
                0, C1 // L, gbody,
                (jnp.int32(N_GRAPHS - 1), tuple(zv for _ in range(HV))))
            for j in range(HV):
                acc[cur, pl.ds(j * L, L)] = acc[cur, pl.ds(j * L, L)] + regs[j]
            return 0
        lax.fori_loop(0, n_i, chunk_body, 0)

        dst = slo_out if half == 0 else shi_out
        pltpu.sync_copy(acc, dst.at[pl.ds(out_base, N_GRAPHS)])
        if half == 0:
            for s in range(N_GRAPHS // 32):
                def srow(r, _):
                    stage_v[r, pl.ds(0, L)] = jnp.full((L,), cnt_s[s * 32 + r])
                    return 0
                lax.fori_loop(0, 32, srow, 0)
                pltpu.sync_copy(
                    stage_v, cnt_out.at[pl.ds(out_base + s * 32, 32)])


TCB = 1000


def _coef_body(slo_ref, shi_ref, cnt_ref, w_ref, x_ref, b_ref, o_ref, tg_v):
    i = pl.program_id(0)

    @pl.when(i == 0)
    def _():
        slo = jnp.sum(slo_ref[...].reshape(NW, N_GRAPHS, HF), axis=0)
        shi = jnp.sum(shi_ref[...].reshape(NW, N_GRAPHS, HF), axis=0)
        c = jnp.sum(cnt_ref[...].reshape(NW, N_GRAPHS, L),
                    axis=(0, 2), keepdims=False).reshape(N_GRAPHS, 1) / L
        s = jnp.concatenate([slo, shi], axis=1)
        mean = s / jnp.clip(c, 1.0, None)
        tg_v[...] = jnp.tanh(
            jnp.dot(mean, w_ref[...], preferred_element_type=jnp.float32))

    xb = x_ref[...]
    gids = lax.broadcasted_iota(jnp.int32, (TCB, N_GRAPHS), 1)
    oh = (gids == b_ref[0, 0, :].reshape(TCB, 1)).astype(jnp.float32)
    tgn = lax.dot_general(oh, tg_v[...],
                          dimension_numbers=(((1,), (0,)), ((), ())),
                          preferred_element_type=jnp.float32)
    d = jnp.sum(xb * tgn, axis=1, keepdims=True)
    coef = 1.0 / (1.0 + jnp.exp(d * -10.0))
    o_ref[...] = coef * xb


def _coef_stage(slo, shi, cnt, w, x, b3):
    grid = N_NODES // TCB
    return pl.pallas_call(
        _coef_body,
        grid=(grid,),
        in_specs=[
            pl.BlockSpec((NW * N_GRAPHS, HF), lambda i: (0, 0)),
            pl.BlockSpec((NW * N_GRAPHS, HF), lambda i: (0, 0)),
            pl.BlockSpec((NW * N_GRAPHS, L), lambda i: (0, 0)),
            pl.BlockSpec((FEAT, FEAT), lambda i: (0, 0)),
            pl.BlockSpec((TCB, FEAT), lambda i: (i, 0)),
            pl.BlockSpec((1, 1, TCB), lambda i: (i, 0, 0)),
        ],
        out_specs=pl.BlockSpec((TCB, FEAT), lambda i: (i, 0)),
        out_shape=jax.ShapeDtypeStruct((N_NODES, FEAT), jnp.float32),
        scratch_shapes=[pltpu.VMEM((N_GRAPHS, FEAT), jnp.float32)],
    )(slo, shi, cnt, w, x, b3)


@functools.partial(
    pl.kernel,
    out_type=[
        jax.ShapeDtypeStruct((NW * N_GRAPHS, HF), jnp.float32),
        jax.ShapeDtypeStruct((NW * N_GRAPHS, HF), jnp.float32),
    ],
    mesh=_mesh,
    scratch_types=[
        pltpu.VMEM((C1,), jnp.int32),
        pltpu.VMEM((C1, HF), jnp.float32),
        pltpu.VMEM((N_GRAPHS, HF), jnp.float32),
    ],
)
def _wsum_sc(w_hbm, b_hbm, olo_out, ohi_out, idx_v, x_v, acc):
    cid = lax.axis_index("c")
    sid = lax.axis_index("s")
    wid = sid * NC + cid
    n_i = (NCH1 - wid + NW - 1) // NW
    out_base = wid * N_GRAPHS

    for half in range(2):
        _zero_acc(acc, N_GRAPHS, HV)

        zv = jnp.zeros((L,), jnp.float32)

        def chunk_body(i, _):
            base = (wid + i * NW) * C1
            pltpu.sync_copy(b_hbm.at[pl.ds(base, C1)], idx_v)
            pltpu.sync_copy(
                w_hbm.at[pl.ds(base, C1), pl.ds(half * HF, HF)], x_v)

            def gbody(g, carry):
                cur = carry[0]
                regs = carry[1]
                bvec = idx_v[pl.ds(g * L, L)]
                b0 = bvec[0]
                b15 = bvec[L - 1]
                fast = jnp.logical_and(b0 == cur, b0 == b15)

                gsum = []
                for j in range(HV):
                    vals = [x_v[g * L + l, pl.ds(j * L, L)] for l in range(L)]
                    while len(vals) > 1:
                        vals = [vals[k] + vals[k + 1]
                                for k in range(0, len(vals), 2)]
                    gsum.append(vals[0])

                @pl.when(jnp.logical_not(fast))
                def _():
                    for j in range(HV):
                        acc[cur, pl.ds(j * L, L)] = (
                            acc[cur, pl.ds(j * L, L)] + regs[j])
                    for l in range(L):
                        n = g * L + l
                        b = bvec[l]
                        for j in range(HV):
                            acc[b, pl.ds(j * L, L)] = (
                                acc[b, pl.ds(j * L, L)]
                                + x_v[n, pl.ds(j * L, L)])

                new_regs = tuple(
                    jnp.where(fast, regs[j] + gsum[j], zv) for j in range(HV))
                new_cur = jnp.where(fast, cur, b15)
                return (new_cur, new_regs)

            cur, regs = lax.fori_loop(
                0, C1 // L, gbody,
                (jnp.int32(N_GRAPHS - 1), tuple(zv for _ in range(HV))))
            for j in range(HV):
                acc[cur, pl.ds(j * L, L)] = acc[cur, pl.ds(j * L, L)] + regs[j]
            return 0
        lax.fori_loop(0, n_i, chunk_body, 0)

        dst = olo_out if half == 0 else ohi_out
        pltpu.sync_copy(acc, dst.at[pl.ds(out_base, N_GRAPHS)])


def _combine_body(olo_ref, ohi_ref, o_ref):
    olo = jnp.sum(olo_ref[...].reshape(NW, N_GRAPHS, HF), axis=0)
    ohi = jnp.sum(ohi_ref[...].reshape(NW, N_GRAPHS, HF), axis=0)
    o_ref[...] = jnp.concatenate([olo, ohi], axis=1)


def _combine(olo, ohi):
    return pl.pallas_call(
        _combine_body,
        out_shape=jax.ShapeDtypeStruct((N_GRAPHS, FEAT), jnp.float32),
    )(olo, ohi)


def kernel(x, batch, W):
    b32 = batch.astype(jnp.int32)
    slo, shi, cnt = _seg_sums_sc(x, b32)
    b3 = b32.reshape(N_NODES // TCB, 1, TCB)
    wrows = _coef_stage(slo, shi, cnt, W, x, b3)
    olo, ohi = _wsum_sc(wrows, b32)
    return _combine(olo, ohi)
